# baseline (speedup 1.0000x reference)
import functools

import jax
import jax.numpy as jnp
from jax import lax
from jax.experimental import pallas as pl
from jax.experimental.pallas import tpu as pltpu
from jax.experimental.pallas import tpu_sc as plsc

B = 4096
T = 200
D = 64
NW = 32
ROWS_W = B // NW
HALF = T // 2
NBUF = 4
GROUPS = D // 16


def _make_sc_call():
  mesh = plsc.VectorSubcoreMesh(core_axis_name="c", subcore_axis_name="s")
  scratch = (
      [pltpu.VMEM((2 * ROWS_W, HALF), jnp.int32)]
      + [pltpu.VMEM((T, D), jnp.float32)]
      + [pltpu.VMEM((T, D), jnp.float32)] * NBUF
      + [pltpu.SemaphoreType.DMA] * NBUF
      + [pltpu.SemaphoreType.DMA] * NBUF
  )

  @functools.partial(
      pl.kernel,
      out_type=jax.ShapeDtypeStruct((B * T, D), jnp.float32),
      mesh=mesh,
      scratch_types=scratch,
      compiler_params=pltpu.CompilerParams(use_tc_tiling_on_sc=False),
  )
  def sc_embed(ids_hbm, tok_hbm, pos_hbm, out_hbm, idx_v, pos_v, *rest):
    bufs = rest[:NBUF]
    gsem = rest[NBUF:2 * NBUF]
    osem = rest[2 * NBUF:]

    wid = lax.axis_index("s") * 2 + lax.axis_index("c")
    out_base = wid * (ROWS_W * T)

    pltpu.sync_copy(ids_hbm.at[wid], idx_v)
    pltpu.sync_copy(pos_hbm, pos_v)

    def gather_start(r, b):
      pltpu.async_copy(
          tok_hbm.at[idx_v.at[2 * r]], bufs[b].at[pl.ds(0, HALF)], gsem[b])
      pltpu.async_copy(
          tok_hbm.at[idx_v.at[2 * r + 1]], bufs[b].at[pl.ds(HALF, HALF)],
          gsem[b])

    def gather_wait(r, b):
      pltpu.make_async_copy(
          tok_hbm.at[idx_v.at[2 * r]], bufs[b].at[pl.ds(0, HALF)],
          gsem[b]).wait()
      pltpu.make_async_copy(
          tok_hbm.at[idx_v.at[2 * r + 1]], bufs[b].at[pl.ds(HALF, HALF)],
          gsem[b]).wait()

    def out_ref(r):
      return out_hbm.at[pl.ds(out_base + r * T, T)]

    def add_pos(b):
      @plsc.parallel_loop(0, T, unroll=8)
      def _(t):
        for g in range(GROUPS):
          sl = pl.ds(g * 16, 16)
          plsc.addupdate(bufs[b].at[t, sl], pos_v[t, sl])

    def process(r, b):
      gather_wait(r, b)
      add_pos(b)
      pltpu.async_copy(bufs[b], out_ref(r), osem[b])

    def refill(r, r_next, b):
      pltpu.make_async_copy(bufs[b], out_ref(r), osem[b]).wait()
      gather_start(r_next, b)

    for b in range(NBUF):
      gather_start(b, b)

    n_rounds = ROWS_W // NBUF

    def round_body(g, carry):
      for b in range(NBUF):
        process(g * NBUF + b, b)
      for b in range(NBUF):
        refill(g * NBUF + b, (g + 1) * NBUF + b, b)
      return carry

    lax.fori_loop(0, n_rounds - 1, round_body, 0, unroll=False)

    g = n_rounds - 1
    for b in range(NBUF):
      process(g * NBUF + b, b)
    for b in range(NBUF):
      pltpu.make_async_copy(bufs[b], out_ref(g * NBUF + b), osem[b]).wait()

  return sc_embed


_SC_EMBED = _make_sc_call()


@jax.jit
def kernel(x_ids, tok_weight, pos_weight):
  Bv, Tv = x_ids.shape
  ids = x_ids.astype(jnp.int32).reshape(NW, 2 * ROWS_W, HALF)
  pos = pos_weight[:Tv]
  out = _SC_EMBED(ids, tok_weight, pos)
  return out.reshape(Bv, Tv, D)
